# baseline (speedup 1.0000x reference)
import functools
import numpy as np
import jax
import jax.numpy as jnp
from jax.experimental import pallas as pl
from jax.experimental.pallas import tpu as pltpu

TEMP_INV = np.float32(1.25)
TOPP = np.float32(0.9)
B, N = 64, 100000
NB = 784
NPAD = NB * 128
NB2 = 7
CH0, CH1 = 261, 522
RB = 8
NEG = np.float32(-1e30)
BIG = np.int32(2 ** 30)


def _strips_halves(acc):
    s8 = jnp.zeros((RB, 8), jnp.float32)
    for i in range(16):
        s8 = s8 + acc[:, i * 8:(i + 1) * 8]
    s4 = s8[:, :4] + s8[:, 4:]
    s2 = s4[:, :2] + s4[:, 2:]
    return s2[:, :1] + s2[:, 1:]


def _row_sum_structured(ref):
    def chunk(lo, hi):
        def body(j, acc):
            return acc + ref[:, j, :]
        return jax.lax.fori_loop(lo, hi, body, jnp.zeros((RB, 128), jnp.float32))
    acc = (chunk(0, CH0) + chunk(CH0, CH1)) + chunk(CH1, NB)
    return _strips_halves(acc)


def _k1_body(in_ref, out_ref, e_ref):
    def mx(j, acc):
        return jnp.maximum(acc, in_ref[:, j, :] * TEMP_INV)
    m = jax.lax.fori_loop(0, NB, mx, jnp.full((RB, 128), NEG, jnp.float32))
    m = jnp.max(m, axis=1, keepdims=True)

    def ex(j, carry):
        e_ref[:, j, :] = jnp.exp(in_ref[:, j, :] * TEMP_INV - m)
        return carry
    jax.lax.fori_loop(0, NB, ex, 0)

    z = _row_sum_structured(e_ref)
    r = np.float32(1.0) / z

    def pr(j, carry):
        out_ref[:, j, :] = e_ref[:, j, :] * r
        return carry
    jax.lax.fori_loop(0, NB, pr, 0)


@jax.jit
def _k1(lp):
    return pl.pallas_call(
        _k1_body,
        grid=(B // RB,),
        in_specs=[pl.BlockSpec((RB, NB, 128), lambda i: (i, 0, 0))],
        out_specs=pl.BlockSpec((RB, NB, 128), lambda i: (i, 0, 0)),
        out_shape=jax.ShapeDtypeStruct((B, NB, 128), jnp.float32),
        scratch_shapes=[pltpu.VMEM((RB, NB, 128), jnp.float32)],
    )(lp)


def _scan_carry(cs_ref, l2_ref):
    bs = cs_ref[:, 127, :]
    zpad = jnp.zeros((RB, 896 - NB), jnp.float32)
    bsp = jnp.concatenate([bs, zpad], axis=1)

    acc = jnp.zeros((RB, 1), jnp.float32)
    for idx in range(896):
        if idx % 128 == 0:
            acc = jnp.zeros((RB, 1), jnp.float32)
        acc = acc + bsp[:, idx:idx + 1]
        l2_ref[:, idx:idx + 1] = acc

    c = jnp.zeros((RB, 1), jnp.float32)
    carries = []
    for k in range(NB2):
        carries.append(c)
        c = c + l2_ref[:, k * 128 + 127:k * 128 + 128]

    parts = []
    for k in range(NB2):
        parts.append(l2_ref[:, k * 128:(k + 1) * 128] + carries[k])
    full2 = jnp.concatenate(parts, axis=1)

    carry_ex = jnp.concatenate(
        [jnp.zeros((RB, 1), jnp.float32), full2[:, :NB - 1]], axis=1)
    return carry_ex


def _k2a_body(tr_ref, psz_ref, cs_ref, l2_ref):
    cs_ref[:, 0, :] = tr_ref[:, 0, :]

    def sc(s, carry):
        cs_ref[:, s, :] = cs_ref[:, s - 1, :] + tr_ref[:, s, :]
        return carry
    jax.lax.fori_loop(1, 128, sc, 0)

    carry_ex = _scan_carry(cs_ref, l2_ref)

    def fin(s, carry):
        ps = tr_ref[:, s, :]
        csv = cs_ref[:, s, :] + carry_ex
        keep = (csv - ps) <= TOPP
        psz_ref[:, s, :] = jnp.where(keep, ps, np.float32(0.0))
        return carry
    jax.lax.fori_loop(0, 128, fin, 0)


@jax.jit
def _k2a(tr):
    return pl.pallas_call(
        _k2a_body,
        grid=(B // RB,),
        in_specs=[pl.BlockSpec((RB, 128, NB), lambda i: (i, 0, 0))],
        out_specs=pl.BlockSpec((RB, 128, NB), lambda i: (i, 0, 0)),
        out_shape=jax.ShapeDtypeStruct((B, 128, NB), jnp.float32),
        scratch_shapes=[pltpu.VMEM((RB, 128, NB), jnp.float32),
                        pltpu.VMEM((RB, 896), jnp.float32)],
    )(tr)


def _lane_prefix_incl(x):
    acc = x
    for d in (1, 2, 4, 8, 16, 32, 64):
        sh = jnp.concatenate(
            [jnp.zeros((RB, d), jnp.int32), acc[:, :128 - d]], axis=1)
        acc = acc + sh
    return acc


def _k2b_body(psz_tr_ref, psz_nat_ref, srt_tr_ref, probs_nat_ref, u_ref,
              tok_ref, cs_ref, l2_ref):
    s_val = _row_sum_structured(psz_nat_ref)
    r_s = np.float32(1.0) / s_val

    cs_ref[:, 0, :] = psz_tr_ref[:, 0, :] * r_s

    def sc(s, carry):
        cs_ref[:, s, :] = cs_ref[:, s - 1, :] + psz_tr_ref[:, s, :] * r_s
        return carry
    jax.lax.fori_loop(1, 128, sc, 0)

    carry_ex = _scan_carry(cs_ref, l2_ref)

    u = u_ref[:, :1]
    tcol = jax.lax.broadcasted_iota(jnp.int32, (RB, NB), 1)

    def cnt(s, acc):
        cdfv = cs_ref[:, s, :] + carry_ex
        valid = (tcol * 128 + s) < N
        return acc + jnp.sum(
            jnp.where((cdfv < u) & valid, np.int32(1), np.int32(0)),
            axis=1, keepdims=True)
    sel = jax.lax.fori_loop(0, 128, cnt, jnp.zeros((RB, 1), jnp.int32))
    sel = jnp.minimum(sel, np.int32(N - 1))

    def vfind(s, vmax):
        flat = tcol * 128 + s
        vals = srt_tr_ref[:, s, :]
        hit = flat == sel
        return jnp.maximum(
            vmax,
            jnp.max(jnp.where(hit, vals, np.float32(-1.0)), axis=1,
                    keepdims=True))
    v = jax.lax.fori_loop(0, 128, vfind,
                          jnp.full((RB, 1), np.float32(-1.0)))

    def gtc(s, acc):
        valid = (tcol * 128 + s) < N
        gt = (srt_tr_ref[:, s, :] > v) & valid
        return acc + jnp.sum(jnp.where(gt, np.int32(1), np.int32(0)),
                             axis=1, keepdims=True)
    gt_n = jax.lax.fori_loop(0, 128, gtc, jnp.zeros((RB, 1), jnp.int32))
    rtie = sel - gt_n

    lane = jax.lax.broadcasted_iota(jnp.int32, (RB, 128), 1)

    def tk(j, carry):
        tokmin, base = carry
        pv = probs_nat_ref[:, j, :]
        flat = j * 128 + lane
        match = (pv == v) & (flat < N)
        mi = jnp.where(match, np.int32(1), np.int32(0))
        incl = _lane_prefix_incl(mi)
        exc = base + incl - mi
        cond = match & (exc == rtie)
        cand = jnp.min(jnp.where(cond, flat, BIG), axis=1, keepdims=True)
        return (jnp.minimum(tokmin, cand), base + incl[:, 127:128])
    tokmin, _ = jax.lax.fori_loop(
        0, NB, tk,
        (jnp.full((RB, 1), BIG, jnp.int32), jnp.zeros((RB, 1), jnp.int32)))
    tok_ref[:, :] = tokmin


@jax.jit
def _k2b(psz_tr, psz_nat, srt_tr, probs_nat, u2):
    return pl.pallas_call(
        _k2b_body,
        grid=(B // RB,),
        in_specs=[pl.BlockSpec((RB, 128, NB), lambda i: (i, 0, 0)),
                  pl.BlockSpec((RB, NB, 128), lambda i: (i, 0, 0)),
                  pl.BlockSpec((RB, 128, NB), lambda i: (i, 0, 0)),
                  pl.BlockSpec((RB, NB, 128), lambda i: (i, 0, 0)),
                  pl.BlockSpec((RB, 1), lambda i: (i, 0))],
        out_specs=pl.BlockSpec((RB, 1), lambda i: (i, 0)),
        out_shape=jax.ShapeDtypeStruct((B, 1), jnp.int32),
        scratch_shapes=[pltpu.VMEM((RB, 128, NB), jnp.float32),
                        pltpu.VMEM((RB, 896), jnp.float32)],
    )(psz_tr, psz_nat, srt_tr, probs_nat, u2)


def kernel(logits, u):
    lp = jnp.pad(logits, ((0, 0), (0, NPAD - N)),
                 constant_values=NEG).reshape(B, NB, 128)
    probs_nat = _k1(lp)
    probs = probs_nat.reshape(B, NPAD)[:, :N]
    sd = jnp.sort(probs, axis=-1)[:, ::-1]
    nat = jnp.pad(sd, ((0, 0), (0, NPAD - N))).reshape(B, NB, 128)
    tr = jnp.transpose(nat, (0, 2, 1))
    psz_tr = _k2a(tr)
    psz_nat = jnp.transpose(psz_tr, (0, 2, 1))
    tok = _k2b(psz_tr, psz_nat, tr, probs_nat, u.reshape(B, 1))
    return tok
